# Initial kernel scaffold; baseline (speedup 1.0000x reference)
import jax
import jax.numpy as jnp
from jax.experimental import pallas as pl


def kernel(x, edge_index, eps_noise, gl_weight, W1, b1, W2, b2, cW1, cb1, cW2, cb2):
    raise NotImplementedError("write your pallas kernel here")


# speedup vs baseline: 3.7654x; 3.7654x over previous
import functools

import jax
import jax.numpy as jnp
from jax import lax
from jax.experimental import pallas as pl
from jax.experimental.pallas import tpu as pltpu
from jax.experimental.pallas import tpu_sc as plsc

G, N, D = 32, 256, 128
P = 4
HID = 256
IB = 128
EPG = 4096
EPSILON = 0.3

_LANES = 16


def _sc_scatter_body(eidx_hbm, out_hbm, rows_v, cols_v, adj_v):
    wid = lax.axis_index("s") * 2 + lax.axis_index("c")

    pltpu.sync_copy(eidx_hbm.at[0, wid], rows_v)
    pltpu.sync_copy(eidx_hbm.at[1, wid], cols_v)

    zeros16 = jnp.zeros((_LANES,), jnp.float32)

    def zbody(i, carry):
        adj_v[pl.ds(i * _LANES, _LANES)] = zeros16
        return carry

    lax.fori_loop(0, (N * N) // _LANES, zbody, 0)

    ones16 = jnp.full((_LANES,), 1.0, jnp.float32)

    def ebody(i, carry):
        r = rows_v[pl.ds(i * _LANES, _LANES)]
        c = cols_v[pl.ds(i * _LANES, _LANES)]
        plsc.store_scatter(adj_v, [r * N + c], ones16)
        return carry

    lax.fori_loop(0, EPG // _LANES, ebody, 0)

    pltpu.sync_copy(adj_v, out_hbm.at[wid])


def _raw_mask_sc(edge_index):
    mesh = plsc.VectorSubcoreMesh(core_axis_name="c", subcore_axis_name="s")
    fn = pl.kernel(
        _sc_scatter_body,
        out_type=jax.ShapeDtypeStruct((G, N * N), jnp.float32),
        mesh=mesh,
        scratch_types=[
            pltpu.VMEM((EPG,), jnp.int32),
            pltpu.VMEM((EPG,), jnp.int32),
            pltpu.VMEM((N * N,), jnp.float32),
        ],
        compiler_params=pltpu.CompilerParams(needs_layout_passes=False),
    )
    return fn(edge_index)


def _tc_body(x_ref, raw_ref, gl_ref, w1_ref, b1_ref, w2_ref, b2_ref,
             eps_ref, cw1_ref, cb1_ref, cw2_ref, cb2_ref,
             mu_ref, std_ref, logit_ref):
    xg = x_ref[...]
    raw = raw_ref[0]
    eps = eps_ref[0]

    parts = []
    for p in range(P):
        wp = gl_ref[p:p + 1, :]
        ex = xg * wp
        q = jnp.sum(ex * ex, axis=1, keepdims=True)
        rinv = 1.0 / (jnp.sqrt(q) + 1e-12)
        parts.append(ex * rinv)
    b_mat = jnp.concatenate(parts, axis=1)
    att = lax.dot_general(b_mat, b_mat, (((1,), (1,)), ((), ())),
                          preferred_element_type=jnp.float32) * (1.0 / P)

    row = lax.broadcasted_iota(jnp.int32, (N, N), 0)
    col = lax.broadcasted_iota(jnp.int32, (N, N), 1)
    a = ((att > EPSILON) | (raw > 0.0)).astype(jnp.float32) \
        + (row == col).astype(jnp.float32)
    deg = jnp.sum(a, axis=1, keepdims=True)
    dinv = 1.0 / jnp.sqrt(deg)

    xw1 = jnp.dot(xg, w1_ref[...], preferred_element_type=jnp.float32)
    h = dinv * jnp.dot(a, dinv * xw1, preferred_element_type=jnp.float32)
    h = jnp.maximum(h + b1_ref[...], 0.0)
    hw2 = jnp.dot(h, w2_ref[...], preferred_element_type=jnp.float32)
    outg = dinv * jnp.dot(a, dinv * hw2, preferred_element_type=jnp.float32)
    outg = outg + b2_ref[...]

    ge = jnp.sum(outg, axis=0, keepdims=True) * (1.0 / N)
    mu = ge[:, :IB]
    s = ge[:, IB:] - float(IB)
    std = jnp.maximum(s, 0.0) + jnp.log1p(jnp.exp(-jnp.abs(s)))
    zn = mu + eps * std

    hc = jnp.dot(zn, cw1_ref[...], preferred_element_type=jnp.float32)
    hc = jnp.maximum(hc + cb1_ref[...], 0.0)
    logits = jnp.dot(hc, cw2_ref[...], preferred_element_type=jnp.float32) \
        + cb2_ref[...]

    mu_ref[0] = mu
    std_ref[0] = std
    logit_ref[0] = logits


def _tc_call(x, raw, gl_weight, W1, b1, W2, b2, eps_noise, cW1, cb1, cW2p, cb2p):
    full = lambda shape: pl.BlockSpec(shape, lambda g: (0,) * len(shape))
    per_g_row = pl.BlockSpec((1, 1, IB), lambda g: (g, 0, 0))
    return pl.pallas_call(
        _tc_body,
        grid=(G,),
        in_specs=[
            pl.BlockSpec((N, D), lambda g: (g, 0)),
            pl.BlockSpec((1, N, N), lambda g: (g, 0, 0)),
            full((P, D)),
            full((D, HID)),
            full((1, HID)),
            full((HID, 2 * IB)),
            full((1, 2 * IB)),
            pl.BlockSpec((1, 1, IB), lambda g: (g, 0, 0)),
            full((IB, IB)),
            full((1, IB)),
            full((IB, IB)),
            full((1, IB)),
        ],
        out_specs=[per_g_row, per_g_row, per_g_row],
        out_shape=[jax.ShapeDtypeStruct((G, 1, IB), jnp.float32)] * 3,
    )(x, raw, gl_weight, W1, b1, W2, b2,
      eps_noise.reshape(G, 1, IB), cW1, cb1, cW2p, cb2p)


def kernel(x, edge_index, eps_noise, gl_weight, W1, b1, W2, b2, cW1, cb1, cW2, cb2):
    raw = _raw_mask_sc(edge_index).reshape(G, N, N)
    ncls = cW2.shape[1]
    cW2p = jnp.pad(cW2, ((0, 0), (0, IB - ncls)))
    cb2p = jnp.pad(cb2, (0, IB - ncls))
    mu, std, logits_pad = _tc_call(
        x, raw, gl_weight, W1, b1.reshape(1, HID), W2, b2.reshape(1, 2 * IB),
        eps_noise, cW1, cb1.reshape(1, IB), cW2p, cb2p.reshape(1, IB))
    return (mu.reshape(G, IB), std.reshape(G, IB),
            logits_pad.reshape(G, IB)[:, :ncls])


# speedup vs baseline: 3.9760x; 1.0559x over previous
import functools

import jax
import jax.numpy as jnp
from jax import lax
from jax.experimental import pallas as pl
from jax.experimental.pallas import tpu as pltpu
from jax.experimental.pallas import tpu_sc as plsc

G, N, D = 32, 256, 128
P = 4
HID = 256
IB = 128
EPG = 4096
EPSILON = 0.3

_LANES = 16
_GPB = 2


def _sc_scatter_body(eidx_hbm, out_hbm, rows_v, cols_v, adj_v):
    wid = lax.axis_index("s") * 2 + lax.axis_index("c")

    pltpu.sync_copy(eidx_hbm.at[0, wid], rows_v)
    pltpu.sync_copy(eidx_hbm.at[1, wid], cols_v)

    zeros16 = jnp.zeros((_LANES,), jnp.float32)

    @plsc.parallel_loop(0, (N * N) // _LANES, unroll=8)
    def _zero(i):
        adj_v[pl.ds(i * _LANES, _LANES)] = zeros16

    ones16 = jnp.full((_LANES,), 1.0, jnp.float32)

    def ebody(i, carry):
        r = rows_v[pl.ds(i * _LANES, _LANES)]
        c = cols_v[pl.ds(i * _LANES, _LANES)]
        plsc.store_scatter(adj_v, [r * N + c], ones16)
        return carry

    lax.fori_loop(0, EPG // _LANES, ebody, 0, unroll=4)

    pltpu.sync_copy(adj_v, out_hbm.at[wid])


def _raw_mask_sc(edge_index):
    mesh = plsc.VectorSubcoreMesh(core_axis_name="c", subcore_axis_name="s")
    fn = pl.kernel(
        _sc_scatter_body,
        out_type=jax.ShapeDtypeStruct((G, N * N), jnp.float32),
        mesh=mesh,
        scratch_types=[
            pltpu.VMEM((EPG,), jnp.int32),
            pltpu.VMEM((EPG,), jnp.int32),
            pltpu.VMEM((N * N,), jnp.float32),
        ],
        compiler_params=pltpu.CompilerParams(needs_layout_passes=False),
    )
    return fn(edge_index)


def _per_graph(xg, raw, eps, gl_ref, gl2t_ref, w1_ref, b1_ref, w2_ref, b2_ref,
               cw1_ref, cb1_ref, cw2_ref, cb2_ref, ones_ref):
    q = jnp.dot(xg * xg, gl2t_ref[...], preferred_element_type=jnp.float32)
    rinv = 1.0 / (jnp.sqrt(q) + 1e-12)
    parts = []
    for p in range(P):
        wp = gl_ref[p:p + 1, :]
        parts.append(xg * wp * rinv[:, p:p + 1])
    b_mat = jnp.concatenate(parts, axis=1)
    att = lax.dot_general(b_mat, b_mat, (((1,), (1,)), ((), ())),
                          preferred_element_type=jnp.float32) * (1.0 / P)

    row = lax.broadcasted_iota(jnp.int32, (N, N), 0)
    col = lax.broadcasted_iota(jnp.int32, (N, N), 1)
    a = ((att > EPSILON) | (raw > 0.0)).astype(jnp.float32) \
        + (row == col).astype(jnp.float32)
    deg = jnp.dot(a, ones_ref[...], preferred_element_type=jnp.float32)
    dinv = 1.0 / jnp.sqrt(deg[:, 0:1])

    xw1 = jnp.dot(xg, w1_ref[...], preferred_element_type=jnp.float32)
    h = dinv * jnp.dot(a, dinv * xw1, preferred_element_type=jnp.float32)
    h = jnp.maximum(h + b1_ref[...], 0.0)
    hw2 = jnp.dot(h, w2_ref[...], preferred_element_type=jnp.float32)
    outg = dinv * jnp.dot(a, dinv * hw2, preferred_element_type=jnp.float32)
    outg = outg + b2_ref[...]

    ge = jnp.sum(outg, axis=0, keepdims=True) * (1.0 / N)
    mu = ge[:, :IB]
    s = ge[:, IB:] - float(IB)
    std = jnp.maximum(s, 0.0) + jnp.log1p(jnp.exp(-jnp.abs(s)))
    zn = mu + eps * std

    hc = jnp.dot(zn, cw1_ref[...], preferred_element_type=jnp.float32)
    hc = jnp.maximum(hc + cb1_ref[...], 0.0)
    logits = jnp.dot(hc, cw2_ref[...], preferred_element_type=jnp.float32) \
        + cb2_ref[...]
    return mu, std, logits


def _tc_body(x_ref, raw_ref, gl_ref, gl2t_ref, w1_ref, b1_ref, w2_ref, b2_ref,
             eps_ref, cw1_ref, cb1_ref, cw2_ref, cb2_ref, ones_ref,
             mu_ref, std_ref, logit_ref):
    for j in range(_GPB):
        xg = x_ref[pl.ds(j * N, N), :]
        raw = raw_ref[j]
        eps = eps_ref[j]
        mu, std, logits = _per_graph(
            xg, raw, eps, gl_ref, gl2t_ref, w1_ref, b1_ref, w2_ref, b2_ref,
            cw1_ref, cb1_ref, cw2_ref, cb2_ref, ones_ref)
        mu_ref[j] = mu
        std_ref[j] = std
        logit_ref[j] = logits


def _tc_call(x, raw, gl_weight, gl2t, W1, b1, W2, b2, eps_noise,
             cW1, cb1, cW2p, cb2p, ones_col):
    full = lambda shape: pl.BlockSpec(shape, lambda g: (0,) * len(shape))
    per_g_row = pl.BlockSpec((_GPB, 1, IB), lambda g: (g, 0, 0))
    return pl.pallas_call(
        _tc_body,
        grid=(G // _GPB,),
        in_specs=[
            pl.BlockSpec((_GPB * N, D), lambda g: (g, 0)),
            pl.BlockSpec((_GPB, N, N), lambda g: (g, 0, 0)),
            full((P, D)),
            full((D, 8)),
            full((D, HID)),
            full((1, HID)),
            full((HID, 2 * IB)),
            full((1, 2 * IB)),
            pl.BlockSpec((_GPB, 1, IB), lambda g: (g, 0, 0)),
            full((IB, IB)),
            full((1, IB)),
            full((IB, IB)),
            full((1, IB)),
            full((N, 8)),
        ],
        out_specs=[per_g_row, per_g_row, per_g_row],
        out_shape=[jax.ShapeDtypeStruct((G, 1, IB), jnp.float32)] * 3,
    )(x, raw, gl_weight, gl2t, W1, b1, W2, b2,
      eps_noise.reshape(G, 1, IB), cW1, cb1, cW2p, cb2p, ones_col)


def kernel(x, edge_index, eps_noise, gl_weight, W1, b1, W2, b2, cW1, cb1, cW2, cb2):
    raw = _raw_mask_sc(edge_index).reshape(G, N, N)
    ncls = cW2.shape[1]
    cW2p = jnp.pad(cW2, ((0, 0), (0, IB - ncls)))
    cb2p = jnp.pad(cb2, (0, IB - ncls))
    gl2t = jnp.pad((gl_weight * gl_weight).T, ((0, 0), (0, 8 - P)))
    ones_col = jnp.ones((N, 8), jnp.float32)
    mu, std, logits_pad = _tc_call(
        x, raw, gl_weight, gl2t, W1, b1.reshape(1, HID), W2,
        b2.reshape(1, 2 * IB), eps_noise, cW1, cb1.reshape(1, IB), cW2p,
        cb2p.reshape(1, IB), ones_col)
    return (mu.reshape(G, IB), std.reshape(G, IB),
            logits_pad.reshape(G, IB)[:, :ncls])
